# baseline (speedup 1.0000x reference)
import functools

import jax
import jax.numpy as jnp
from jax import lax
from jax.experimental import pallas as pl
from jax.experimental.pallas import tpu as pltpu
from jax.experimental.pallas import tpu_sc as plsc

N = 10000
D = 128
E = 320000
NC, NS = 2, 16
NW = NC * NS
EW = E // NW
CH = 128
NCHUNK = 79
EWP = NCHUNK * CH
NBUF = 3
NPAD = 10240
HN = NPAD // 2
RPH = HN // NS
NDUMP = 64
ACCR = HN + NDUMP

_MESH = plsc.VectorSubcoreMesh(core_axis_name="c", subcore_axis_name="s")


def _chunk_copy(all_v, c, out_v):
    for g in range(CH // 16):
        out_v[pl.ds(g * 16, 16)] = all_v[pl.ds(c * CH + g * 16, 16)]


def _rebase(all_v, c, lo, out_v):
    for g in range(CH // 16):
        v = all_v[pl.ds(c * CH + g * 16, 16)]
        inh = (v >= lo) & (v < lo + HN) & (v < N)
        out_v[pl.ds(g * 16, 16)] = jnp.where(
            inh, v - lo, HN + (v & (NDUMP - 1)))


def _deg_body(dst_hbm, ones_hbm, zeros_hbm, out_hbm,
              dall_v, d2_v, ones_v, acc_sh):
    cid = lax.axis_index("c")
    sid = lax.axis_index("s")
    wid = cid * NS + sid
    base = wid * EWP
    pltpu.sync_copy(ones_hbm, ones_v)
    pltpu.sync_copy(dst_hbm.at[pl.ds(base, EWP)], dall_v)
    for half in range(2):
        lo = half * HN
        pltpu.sync_copy(zeros_hbm, acc_sh.at[pl.ds(sid * RPH, RPH)])
        plsc.subcore_barrier()

        def step(c):
            _rebase(dall_v, c, lo, d2_v)
            pltpu.sync_copy(ones_v, acc_sh.at[d2_v], add=True)

        step(0)

        def pair(c2, carry):
            c = 1 + 2 * c2
            step(c)
            step(c + 1)
            return carry

        lax.fori_loop(0, (NCHUNK - 1) // 2, pair, 0)
        plsc.subcore_barrier()
        pltpu.sync_copy(acc_sh.at[pl.ds(sid * RPH, RPH)],
                        out_hbm.at[cid, half, pl.ds(sid * RPH, RPH)])


_deg_call = pl.kernel(
    _deg_body,
    out_type=jax.ShapeDtypeStruct((NC, 2, HN, D), jnp.float32),
    mesh=_MESH,
    scratch_types=[
        pltpu.VMEM((EWP,), jnp.int32),
        pltpu.VMEM((CH,), jnp.int32),
        pltpu.VMEM((CH, D), jnp.float32),
        pltpu.VMEM_SHARED((ACCR, D), jnp.float32),
    ],
)


def _prop_body(z_hbm, src_hbm, dst_hbm, zeros_hbm, out_hbm,
               sall_v, dall_v, s0_v, s1_v, s2_v, d2_v, r0_v, r1_v, r2_v,
               gs0, gs1, gs2, acc_sh):
    cid = lax.axis_index("c")
    sid = lax.axis_index("s")
    wid = cid * NS + sid
    base = wid * EWP
    S = [s0_v, s1_v, s2_v]
    R, GS = [r0_v, r1_v, r2_v], [gs0, gs1, gs2]
    pltpu.sync_copy(src_hbm.at[pl.ds(base, EWP)], sall_v)
    pltpu.sync_copy(dst_hbm.at[pl.ds(base, EWP)], dall_v)
    for half in range(2):
        lo = half * HN
        pltpu.sync_copy(zeros_hbm, acc_sh.at[pl.ds(sid * RPH, RPH)])
        plsc.subcore_barrier()

        _chunk_copy(sall_v, 0, S[0])
        pltpu.async_copy(z_hbm.at[S[0]], R[0], GS[0])
        _chunk_copy(sall_v, 1, S[1])
        pltpu.async_copy(z_hbm.at[S[1]], R[1], GS[1])

        def step(c, b, prefetch):
            _rebase(dall_v, c, lo, d2_v)
            pltpu.make_async_copy(z_hbm.at[S[b]], R[b], GS[b]).wait()
            if prefetch:
                b2 = (b + 2) % NBUF
                _chunk_copy(sall_v, c + 2, S[b2])
                pltpu.async_copy(z_hbm.at[S[b2]], R[b2], GS[b2])
            pltpu.sync_copy(R[b], acc_sh.at[d2_v], add=True)

        step(0, 0, True)

        def triple(c3, carry):
            c = 1 + 3 * c3
            step(c, 1, True)
            step(c + 1, 2, True)
            step(c + 2, 0, True)
            return carry

        lax.fori_loop(0, (NCHUNK - 4) // 3, triple, 0)
        step(NCHUNK - 3, 1, True)
        step(NCHUNK - 2, 2, False)
        step(NCHUNK - 1, 0, False)
        plsc.subcore_barrier()
        pltpu.sync_copy(acc_sh.at[pl.ds(sid * RPH, RPH)],
                        out_hbm.at[cid, half, pl.ds(sid * RPH, RPH)])


_prop_call = pl.kernel(
    _prop_body,
    out_type=jax.ShapeDtypeStruct((NC, 2, HN, D), jnp.float32),
    mesh=_MESH,
    scratch_types=[
        pltpu.VMEM((EWP,), jnp.int32),
        pltpu.VMEM((EWP,), jnp.int32),
        pltpu.VMEM((CH,), jnp.int32),
        pltpu.VMEM((CH,), jnp.int32),
        pltpu.VMEM((CH,), jnp.int32),
        pltpu.VMEM((CH,), jnp.int32),
        pltpu.VMEM((CH, D), jnp.float32),
        pltpu.VMEM((CH, D), jnp.float32),
        pltpu.VMEM((CH, D), jnp.float32),
        pltpu.SemaphoreType.DMA,
        pltpu.SemaphoreType.DMA,
        pltpu.SemaphoreType.DMA,
        pltpu.VMEM_SHARED((ACCR, D), jnp.float32),
    ],
)


_BM = 640


def _dinv(degp):
    return lax.rsqrt(degp[0, 0, :, 0] + degp[1, 0, :, 0] + 1.0)


def _combine(p_ref, z_ref):
    return p_ref[0, 0] + p_ref[1, 0] + z_ref[...]


def _xw_body(x_ref, w_ref, degp_ref, z_ref):
    dinv = _dinv(degp_ref[...])
    z_ref[...] = (x_ref[...] @ w_ref[...]) * dinv[:, None]


def _hidden_body(p_ref, z1_ref, degp_ref, b_ref, z2_ref):
    dinv = _dinv(degp_ref[...])
    p = _combine(p_ref, z1_ref) * dinv[:, None]
    h = jnp.maximum(p + b_ref[...], 0.0)
    z2_ref[...] = h * dinv[:, None]


def _head_body(p_ref, z2_ref, degp_ref, wcat_ref, bcat_ref, mean_ref, std_ref):
    dinv = _dinv(degp_ref[...])
    q = _combine(p_ref, z2_ref) * dinv[:, None]
    o = q @ wcat_ref[...] + bcat_ref[...]
    mean_ref[...] = o[:, :D]
    std_ref[...] = jnp.exp(0.5 * o[:, D:])


_part_spec = pl.BlockSpec((NC, 1, _BM, D), lambda i: (0, i // 8, i % 8, 0))
_row_spec = pl.BlockSpec((_BM, D), lambda i: (i, 0))
_z_shape = jax.ShapeDtypeStruct((NPAD, D), jnp.float32)

_xw_call = pl.pallas_call(
    _xw_body,
    grid=(NPAD // _BM,),
    in_specs=[
        _row_spec,
        pl.BlockSpec((D, D), lambda i: (0, 0)),
        _part_spec,
    ],
    out_specs=_row_spec,
    out_shape=_z_shape,
)

_hidden_call = pl.pallas_call(
    _hidden_body,
    grid=(NPAD // _BM,),
    in_specs=[
        _part_spec,
        _row_spec,
        _part_spec,
        pl.BlockSpec((1, D), lambda i: (0, 0)),
    ],
    out_specs=_row_spec,
    out_shape=_z_shape,
)

_head_call = pl.pallas_call(
    _head_body,
    grid=(NPAD // _BM,),
    in_specs=[
        _part_spec,
        _row_spec,
        _part_spec,
        pl.BlockSpec((D, 2 * D), lambda i: (0, 0)),
        pl.BlockSpec((1, 2 * D), lambda i: (0, 0)),
    ],
    out_specs=[
        pl.BlockSpec((_BM, D), lambda i: (i, 0)),
        pl.BlockSpec((_BM, D), lambda i: (i, 0)),
    ],
    out_shape=[
        jax.ShapeDtypeStruct((N, D), jnp.float32),
        jax.ShapeDtypeStruct((N, D), jnp.float32),
    ],
)


def kernel(x, edge_index, W_bb, b_bb, W_mean, b_mean, W_logvar, b_logvar):
    srcw = edge_index[0].astype(jnp.int32).reshape(NW, EW)
    dstw = edge_index[1].astype(jnp.int32).reshape(NW, EW)
    pad_src = jnp.zeros((NW, EWP - EW), jnp.int32)
    pad_dst = jnp.broadcast_to(
        N + (jnp.arange(EWP - EW, dtype=jnp.int32) % NDUMP),
        (NW, EWP - EW))
    src = jnp.concatenate([srcw, pad_src], axis=1).reshape(-1)
    dst = jnp.concatenate([dstw, pad_dst], axis=1).reshape(-1)
    ones128 = jnp.ones((CH, D), jnp.float32)
    zeros128 = jnp.zeros((RPH, D), jnp.float32)

    degp = _deg_call(dst, ones128, zeros128)
    z1 = _xw_call(x, W_bb, degp)
    p1 = _prop_call(z1, src, dst, zeros128)
    z2 = _hidden_call(p1, z1, degp, b_bb.reshape(1, D))
    p2 = _prop_call(z2, src, dst, zeros128)
    wcat = jnp.concatenate([W_mean, W_logvar], axis=1)
    bcat = jnp.concatenate([b_mean, b_logvar]).reshape(1, 2 * D)
    mean, std = _head_call(p2, z2, degp, wcat, bcat)
    return (mean, std)


# speedup vs baseline: 1.8709x; 1.8709x over previous
import functools

import jax
import jax.numpy as jnp
from jax import lax
from jax.experimental import pallas as pl
from jax.experimental.pallas import tpu as pltpu
from jax.experimental.pallas import tpu_sc as plsc

N = 10000
D = 128
E = 320000
NC, NS = 2, 16
NW = NC * NS
EW = E // NW
CH = 80
NCHUNK = EW // CH
NBUF = 3
NPAD = 10240
HN = NPAD // 2
RPH = HN // NS
NDUMP = 64
ACCR = HN + NDUMP

_MESH = plsc.VectorSubcoreMesh(core_axis_name="c", subcore_axis_name="s")


def _chunk_copy(all_v, c, out_v):
    for g in range(CH // 16):
        out_v[pl.ds(g * 16, 16)] = all_v[pl.ds(c * CH + g * 16, 16)]


def _rebase(all_v, c, lo, out_v):
    for g in range(CH // 16):
        v = all_v[pl.ds(c * CH + g * 16, 16)]
        inh = (v >= lo) & (v < lo + HN) & (v < N)
        out_v[pl.ds(g * 16, 16)] = jnp.where(
            inh, v - lo, HN + (v & (NDUMP - 1)))


def _deg_body(dst_hbm, ones_hbm, zeros_hbm, out_hbm,
              dall_v, d2_v, ones_v, acc_sh):
    cid = lax.axis_index("c")
    sid = lax.axis_index("s")
    wid = cid * NS + sid
    base = wid * EW
    pltpu.sync_copy(ones_hbm, ones_v)
    pltpu.sync_copy(dst_hbm.at[pl.ds(base, EW)], dall_v)
    for half in range(2):
        lo = half * HN
        pltpu.sync_copy(zeros_hbm, acc_sh.at[pl.ds(sid * RPH, RPH)])
        plsc.subcore_barrier()

        def step(c):
            _rebase(dall_v, c, lo, d2_v)
            pltpu.sync_copy(ones_v, acc_sh.at[d2_v], add=True)

        step(0)

        def pair(c2, carry):
            c = 1 + 2 * c2
            step(c)
            step(c + 1)
            return carry

        lax.fori_loop(0, (NCHUNK - 1) // 2, pair, 0)
        plsc.subcore_barrier()
        pltpu.sync_copy(acc_sh.at[pl.ds(sid * RPH, RPH)],
                        out_hbm.at[cid, half, pl.ds(sid * RPH, RPH)])


_deg_call = pl.kernel(
    _deg_body,
    out_type=jax.ShapeDtypeStruct((NC, 2, HN, D), jnp.float32),
    mesh=_MESH,
    scratch_types=[
        pltpu.VMEM((EW,), jnp.int32),
        pltpu.VMEM((CH,), jnp.int32),
        pltpu.VMEM((CH, D), jnp.float32),
        pltpu.VMEM_SHARED((ACCR, D), jnp.float32),
    ],
)


def _prop_body(z_hbm, src_hbm, dst_hbm, zeros_hbm, out_hbm,
               sall_v, dall_v, s0_v, s1_v, s2_v, d2_v, r0_v, r1_v, r2_v,
               gs0, gs1, gs2, acc_sh):
    cid = lax.axis_index("c")
    sid = lax.axis_index("s")
    wid = cid * NS + sid
    base = wid * EW
    S = [s0_v, s1_v, s2_v]
    R, GS = [r0_v, r1_v, r2_v], [gs0, gs1, gs2]
    pltpu.sync_copy(src_hbm.at[pl.ds(base, EW)], sall_v)
    pltpu.sync_copy(dst_hbm.at[pl.ds(base, EW)], dall_v)
    for half in range(2):
        lo = half * HN
        pltpu.sync_copy(zeros_hbm, acc_sh.at[pl.ds(sid * RPH, RPH)])
        plsc.subcore_barrier()

        _chunk_copy(sall_v, 0, S[0])
        pltpu.async_copy(z_hbm.at[S[0]], R[0], GS[0])
        _chunk_copy(sall_v, 1, S[1])
        pltpu.async_copy(z_hbm.at[S[1]], R[1], GS[1])

        def step(c, b, prefetch):
            _rebase(dall_v, c, lo, d2_v)
            pltpu.make_async_copy(z_hbm.at[S[b]], R[b], GS[b]).wait()
            if prefetch:
                b2 = (b + 2) % NBUF
                _chunk_copy(sall_v, c + 2, S[b2])
                pltpu.async_copy(z_hbm.at[S[b2]], R[b2], GS[b2])
            pltpu.sync_copy(R[b], acc_sh.at[d2_v], add=True)

        step(0, 0, True)

        def triple(c3, carry):
            c = 1 + 3 * c3
            step(c, 1, True)
            step(c + 1, 2, True)
            step(c + 2, 0, True)
            return carry

        lax.fori_loop(0, (NCHUNK - 5) // 3, triple, 0)
        step(NCHUNK - 4, 1, True)
        step(NCHUNK - 3, 2, True)
        step(NCHUNK - 2, 0, False)
        step(NCHUNK - 1, 1, False)
        plsc.subcore_barrier()
        pltpu.sync_copy(acc_sh.at[pl.ds(sid * RPH, RPH)],
                        out_hbm.at[cid, half, pl.ds(sid * RPH, RPH)])


_prop_call = pl.kernel(
    _prop_body,
    out_type=jax.ShapeDtypeStruct((NC, 2, HN, D), jnp.float32),
    mesh=_MESH,
    scratch_types=[
        pltpu.VMEM((EW,), jnp.int32),
        pltpu.VMEM((EW,), jnp.int32),
        pltpu.VMEM((CH,), jnp.int32),
        pltpu.VMEM((CH,), jnp.int32),
        pltpu.VMEM((CH,), jnp.int32),
        pltpu.VMEM((CH,), jnp.int32),
        pltpu.VMEM((CH, D), jnp.float32),
        pltpu.VMEM((CH, D), jnp.float32),
        pltpu.VMEM((CH, D), jnp.float32),
        pltpu.SemaphoreType.DMA,
        pltpu.SemaphoreType.DMA,
        pltpu.SemaphoreType.DMA,
        pltpu.VMEM_SHARED((ACCR, D), jnp.float32),
    ],
)


_BM = 640


def _dinv(degp):
    return lax.rsqrt(degp[0, 0, :, 0] + degp[1, 0, :, 0] + 1.0)


def _combine(p_ref, z_ref):
    return p_ref[0, 0] + p_ref[1, 0] + z_ref[...]


def _xw_body(x_ref, w_ref, degp_ref, z_ref):
    dinv = _dinv(degp_ref[...])
    z_ref[...] = (x_ref[...] @ w_ref[...]) * dinv[:, None]


def _hidden_body(p_ref, z1_ref, degp_ref, b_ref, z2_ref):
    dinv = _dinv(degp_ref[...])
    p = _combine(p_ref, z1_ref) * dinv[:, None]
    h = jnp.maximum(p + b_ref[...], 0.0)
    z2_ref[...] = h * dinv[:, None]


def _head_body(p_ref, z2_ref, degp_ref, wcat_ref, bcat_ref, mean_ref, std_ref):
    dinv = _dinv(degp_ref[...])
    q = _combine(p_ref, z2_ref) * dinv[:, None]
    o = q @ wcat_ref[...] + bcat_ref[...]
    mean_ref[...] = o[:, :D]
    std_ref[...] = jnp.exp(0.5 * o[:, D:])


_part_spec = pl.BlockSpec((NC, 1, _BM, D), lambda i: (0, i // 8, i % 8, 0))
_row_spec = pl.BlockSpec((_BM, D), lambda i: (i, 0))
_z_shape = jax.ShapeDtypeStruct((NPAD, D), jnp.float32)

_xw_call = pl.pallas_call(
    _xw_body,
    grid=(NPAD // _BM,),
    in_specs=[
        _row_spec,
        pl.BlockSpec((D, D), lambda i: (0, 0)),
        _part_spec,
    ],
    out_specs=_row_spec,
    out_shape=_z_shape,
)

_hidden_call = pl.pallas_call(
    _hidden_body,
    grid=(NPAD // _BM,),
    in_specs=[
        _part_spec,
        _row_spec,
        _part_spec,
        pl.BlockSpec((1, D), lambda i: (0, 0)),
    ],
    out_specs=_row_spec,
    out_shape=_z_shape,
)

_head_call = pl.pallas_call(
    _head_body,
    grid=(NPAD // _BM,),
    in_specs=[
        _part_spec,
        _row_spec,
        _part_spec,
        pl.BlockSpec((D, 2 * D), lambda i: (0, 0)),
        pl.BlockSpec((1, 2 * D), lambda i: (0, 0)),
    ],
    out_specs=[
        pl.BlockSpec((_BM, D), lambda i: (i, 0)),
        pl.BlockSpec((_BM, D), lambda i: (i, 0)),
    ],
    out_shape=[
        jax.ShapeDtypeStruct((N, D), jnp.float32),
        jax.ShapeDtypeStruct((N, D), jnp.float32),
    ],
)


def kernel(x, edge_index, W_bb, b_bb, W_mean, b_mean, W_logvar, b_logvar):
    src = edge_index[0].astype(jnp.int32)
    dst = edge_index[1].astype(jnp.int32)
    ones128 = jnp.ones((CH, D), jnp.float32)
    zeros128 = jnp.zeros((RPH, D), jnp.float32)

    degp = _deg_call(dst, ones128, zeros128)
    z1 = _xw_call(x, W_bb, degp)
    p1 = _prop_call(z1, src, dst, zeros128)
    z2 = _hidden_call(p1, z1, degp, b_bb.reshape(1, D))
    p2 = _prop_call(z2, src, dst, zeros128)
    wcat = jnp.concatenate([W_mean, W_logvar], axis=1)
    bcat = jnp.concatenate([b_mean, b_logvar]).reshape(1, 2 * D)
    mean, std = _head_call(p2, z2, degp, wcat, bcat)
    return (mean, std)
